# baseline (device time: 29484 ns/iter reference)
import jax
import jax.numpy as jnp
from jax import lax
from jax.experimental import pallas as pl
from jax.experimental.pallas import tpu as pltpu

N_DEV = 4
E_PER_DEV = 4
N_TOK = 1024
D_MODEL = 512
D_FF = 1024
N_EXP = 16
ROWS = N_TOK // N_DEV


def kernel(x, router_W, route_idx, expert_W, shared_W):
    def body(x_ref, rw_ref, idx_ref, ew_ref, sw_ref, out_ref,
             send_ref, recv_ref, send_sems, recv_sems):
        my_pos = lax.axis_index("i")

        barrier_sem = pltpu.get_barrier_semaphore()
        for off in range(1, N_DEV):
            pl.semaphore_signal(
                barrier_sem, inc=1,
                device_id=((my_pos + off) % N_DEV,),
                device_id_type=pl.DeviceIdType.MESH,
            )
        pl.semaphore_wait(barrier_sem, N_DEV - 1)

        ew_bf = ew_ref[:, :, :].reshape(E_PER_DEV * D_MODEL, D_FF).astype(
            jnp.bfloat16)
        sw_bf = sw_ref[:, :].astype(jnp.bfloat16)
        w_mine = jnp.concatenate([ew_bf, sw_bf], axis=0)

        def chunk_partial(c, with_shared):
            rows = pl.ds(c * ROWS, ROWS)
            xc = x_ref[rows, :]
            scores = jnp.dot(xc, rw_ref[:, :],
                             preferred_element_type=jnp.float32)
            scores = scores - jnp.max(scores, axis=-1, keepdims=True)
            es = jnp.exp(scores)
            probs = es / jnp.sum(es, axis=-1, keepdims=True)
            idx = idx_ref[rows, :]
            lanes = lax.broadcasted_iota(jnp.int32, (ROWS, N_EXP), 1)
            sel_prob = jnp.sum(jnp.where(lanes == idx, probs, 0.0),
                               axis=-1, keepdims=True)
            xc_bf = xc.astype(jnp.bfloat16)
            blocks = []
            for e_local in range(E_PER_DEV):
                e_glob = my_pos * E_PER_DEV + e_local
                coeff = jnp.where(idx == e_glob, sel_prob, 0.0)
                blocks.append(xc_bf * coeff.astype(jnp.bfloat16))
            if with_shared:
                blocks.append(xc_bf)
            x_cat = jnp.concatenate(blocks, axis=1)
            return jnp.dot(x_cat, w_mine if with_shared else ew_bf,
                           preferred_element_type=jnp.float32)

        rdmas = []
        for k in (1, 0, 2):
            peer = (my_pos + 1 + k) % N_DEV
            send_ref[k, :, :] = chunk_partial(peer, False).astype(
                jnp.bfloat16)
            rdma = pltpu.make_async_remote_copy(
                src_ref=send_ref.at[k],
                dst_ref=recv_ref.at[2 - k],
                send_sem=send_sems.at[k],
                recv_sem=recv_sems.at[2 - k],
                device_id=(peer,),
                device_id_type=pl.DeviceIdType.MESH,
            )
            rdma.start()
            rdmas.append(rdma)

        acc = chunk_partial(my_pos, True)

        for rdma in rdmas:
            rdma.wait_recv()
        for j in range(N_DEV - 1):
            acc = acc + recv_ref[j, :, :].astype(jnp.float32)
        out_ref[:, :] = acc
        for rdma in rdmas:
            rdma.wait_send()

    return pl.pallas_call(
        body,
        out_shape=jax.ShapeDtypeStruct((ROWS, D_FF), jnp.float32),
        in_specs=[pl.BlockSpec(memory_space=pltpu.VMEM)] * 5,
        out_specs=pl.BlockSpec(memory_space=pltpu.VMEM),
        scratch_shapes=[
            pltpu.VMEM((N_DEV - 1, ROWS, D_FF), jnp.bfloat16),
            pltpu.VMEM((N_DEV - 1, ROWS, D_FF), jnp.bfloat16),
            pltpu.SemaphoreType.DMA((N_DEV - 1,)),
            pltpu.SemaphoreType.DMA((N_DEV - 1,)),
        ],
        compiler_params=pltpu.CompilerParams(collective_id=0),
    )(x, router_W, route_idx, expert_W, shared_W)


# device time: 23953 ns/iter; 1.2309x vs baseline; 1.2309x over previous
import jax
import jax.numpy as jnp
from jax import lax
from jax.experimental import pallas as pl
from jax.experimental.pallas import tpu as pltpu

N_DEV = 4
E_PER_DEV = 4
N_TOK = 1024
D_MODEL = 512
D_FF = 1024
N_EXP = 16
ROWS = N_TOK // N_DEV
CAP = 128


def kernel(x, router_W, route_idx, expert_W, shared_W):
    def body(x_ref, rw_ref, idx_ref, ew_hbm, sw_hbm, out_ref,
             ew_vmem, ew_bf, sw_vmem, send_ref, recv_ref,
             copy_sems, send_sems, recv_sems):
        my_pos = lax.axis_index("i")

        ew_copies = []
        for e in range(E_PER_DEV):
            c = pltpu.make_async_copy(ew_hbm.at[e], ew_vmem.at[e],
                                      copy_sems.at[e])
            c.start()
            ew_copies.append(c)
        sw_copy = pltpu.make_async_copy(sw_hbm, sw_vmem,
                                        copy_sems.at[E_PER_DEV])
        sw_copy.start()

        barrier_sem = pltpu.get_barrier_semaphore()
        for off in range(1, N_DEV):
            pl.semaphore_signal(
                barrier_sem, inc=1,
                device_id=((my_pos + off) % N_DEV,),
                device_id_type=pl.DeviceIdType.MESH,
            )
        pl.semaphore_wait(barrier_sem, N_DEV - 1)

        tri_i = lax.broadcasted_iota(jnp.int32, (ROWS, ROWS), 0)
        tri_j = lax.broadcasted_iota(jnp.int32, (ROWS, ROWS), 1)
        tri = jnp.where(tri_j <= tri_i, 1.0, 0.0).astype(jnp.float32)
        cap_iota = lax.broadcasted_iota(jnp.int32, (ROWS, CAP), 1)

        def pack_matrix(idx_c, owner):
            lo = owner * E_PER_DEV
            mine = (idx_c >= lo) & (idx_c < lo + E_PER_DEV)
            mask_f = jnp.where(mine, 1.0, 0.0).astype(jnp.float32)
            rank = jnp.dot(tri, mask_f,
                           preferred_element_type=jnp.float32)
            slot = rank.astype(jnp.int32) - 1
            return jnp.where((cap_iota == slot) & mine, 1.0, 0.0)

        def chunk_inputs(c):
            rows = pl.ds(c * ROWS, ROWS)
            xc = x_ref[rows, :]
            scores = jnp.dot(xc, rw_ref[:, :],
                             preferred_element_type=jnp.float32)
            scores = scores - jnp.max(scores, axis=-1, keepdims=True)
            es = jnp.exp(scores)
            probs = es / jnp.sum(es, axis=-1, keepdims=True)
            idx_c = idx_ref[rows, :]
            lanes = lax.broadcasted_iota(jnp.int32, (ROWS, N_EXP), 1)
            sel_prob = jnp.sum(jnp.where(lanes == idx_c, probs, 0.0),
                               axis=-1, keepdims=True)
            m = pack_matrix(idx_c, my_pos).astype(jnp.bfloat16)
            xs = (xc * sel_prob).astype(jnp.bfloat16)
            xp = lax.dot_general(m, xs, (((0,), (0,)), ((), ())),
                                 preferred_element_type=jnp.float32)
            ip = lax.dot_general(m, idx_c.astype(jnp.bfloat16),
                                 (((0,), (0,)), ((), ())),
                                 preferred_element_type=jnp.float32)
            return xp.astype(jnp.bfloat16), ip, m

        ew_ready = [False] * E_PER_DEV

        def packed_partial(xp, ip):
            part = jnp.zeros((CAP, D_FF), jnp.float32)
            for e in range(E_PER_DEV):
                if not ew_ready[e]:
                    ew_copies[e].wait()
                    ew_bf[e, :, :] = ew_vmem[e, :, :].astype(jnp.bfloat16)
                    ew_ready[e] = True
                e_glob = my_pos * E_PER_DEV + e
                emask = jnp.where(ip == e_glob, 1.0, 0.0).astype(jnp.bfloat16)
                part = part + jnp.dot(xp * emask, ew_bf[e, :, :],
                                      preferred_element_type=jnp.float32)
            return part

        rdmas = []
        for k in (1, 0, 2):
            peer = (my_pos + 1 + k) % N_DEV
            xp, ip, _ = chunk_inputs(peer)
            send_ref[k, :, :] = packed_partial(xp, ip).astype(jnp.bfloat16)
            rdma = pltpu.make_async_remote_copy(
                src_ref=send_ref.at[k],
                dst_ref=recv_ref.at[2 - k],
                send_sem=send_sems.at[k],
                recv_sem=recv_sems.at[2 - k],
                device_id=(peer,),
                device_id_type=pl.DeviceIdType.MESH,
            )
            rdma.start()
            rdmas.append(rdma)

        xp, ip, m_mine = chunk_inputs(my_pos)
        part = packed_partial(xp, ip)
        acc = jnp.dot(m_mine.astype(jnp.float32), part,
                      preferred_element_type=jnp.float32)
        sw_copy.wait()
        my_rows = pl.ds(my_pos * ROWS, ROWS)
        acc = acc + jnp.dot(
            x_ref[my_rows, :].astype(jnp.bfloat16),
            sw_vmem[:, :].astype(jnp.bfloat16),
            preferred_element_type=jnp.float32,
        )

        idx_mine = idx_ref[my_rows, :]
        for j, rdma in ((1, rdmas[0]), (2, rdmas[1]), (0, rdmas[2])):
            rdma.wait_recv()
            sender = (my_pos + 1 + j) % N_DEV
            m_s = pack_matrix(idx_mine, sender).astype(jnp.bfloat16)
            acc = acc + jnp.dot(m_s, recv_ref[j, :, :],
                                preferred_element_type=jnp.float32)
        out_ref[:, :] = acc
        for rdma in rdmas:
            rdma.wait_send()

    return pl.pallas_call(
        body,
        out_shape=jax.ShapeDtypeStruct((ROWS, D_FF), jnp.float32),
        in_specs=[
            pl.BlockSpec(memory_space=pltpu.VMEM),
            pl.BlockSpec(memory_space=pltpu.VMEM),
            pl.BlockSpec(memory_space=pltpu.VMEM),
            pl.BlockSpec(memory_space=pl.ANY),
            pl.BlockSpec(memory_space=pl.ANY),
        ],
        out_specs=pl.BlockSpec(memory_space=pltpu.VMEM),
        scratch_shapes=[
            pltpu.VMEM((E_PER_DEV, D_MODEL, D_FF), jnp.float32),
            pltpu.VMEM((E_PER_DEV, D_MODEL, D_FF), jnp.bfloat16),
            pltpu.VMEM((D_MODEL, D_FF), jnp.float32),
            pltpu.VMEM((N_DEV - 1, CAP, D_FF), jnp.bfloat16),
            pltpu.VMEM((N_DEV - 1, CAP, D_FF), jnp.bfloat16),
            pltpu.SemaphoreType.DMA((E_PER_DEV + 1,)),
            pltpu.SemaphoreType.DMA((N_DEV - 1,)),
            pltpu.SemaphoreType.DMA((N_DEV - 1,)),
        ],
        compiler_params=pltpu.CompilerParams(collective_id=0),
    )(x, router_W, route_idx, expert_W, shared_W)
